# baseline (speedup 1.0000x reference)
import functools

import jax
import jax.numpy as jnp
from jax import lax
from jax.experimental import pallas as pl
from jax.experimental.pallas import tpu as pltpu
from jax.experimental.pallas import tpu_sc as plsc

N = 10000
E = 160000
F_IN = 256
H = 128
G = 128

NW = 32
CH = 64
EPW = 5120
E_PAD = NW * EPW
NCH = EPW // CH
RPT = 624
REM = N - 16 * RPT
REM_OFF = 16 * RPT
N_ACC = N + 8
BLK = 1000
NB = N // BLK

@functools.cache
def _get_sc_agg():
    mesh = plsc.VectorSubcoreMesh(core_axis_name="c", subcore_axis_name="s")

    @functools.partial(
        pl.kernel,
        out_type=jax.ShapeDtypeStruct((2 * N, H), jnp.float32),
        mesh=mesh,
        scratch_types=[
            pltpu.VMEM((NCH // 2, 2 * CH), jnp.int32),
            pltpu.VMEM((NCH, CH), jnp.int32),
            [pltpu.VMEM((CH, H), jnp.float32)] * 4,
            pltpu.VMEM_SHARED((N_ACC, H), jnp.float32),
            [pltpu.SemaphoreType.DMA] * 4,
            [pltpu.SemaphoreType.DMA] * 4,
        ],
    )
    def _sc_segment_sum(h_hbm, src_hbm, dst_hbm, zeros_hbm, out_hbm,
                        idx_s, idx_d, rows, acc, gsem, ssem):
        c = lax.axis_index("c")
        s = lax.axis_index("s")
        w = s * 2 + c
        pltpu.sync_copy(zeros_hbm, acc.at[pl.ds(s * RPT, RPT)])
        pltpu.sync_copy(src_hbm.at[pl.ds(w * (NCH // 2), NCH // 2)], idx_s)
        pltpu.sync_copy(dst_hbm.at[pl.ds(w * NCH, NCH)], idx_d)

        @pl.when(s == 15)
        def _():
            pltpu.sync_copy(zeros_hbm.at[pl.ds(0, REM)],
                            acc.at[pl.ds(REM_OFF, REM)])

        plsc.subcore_barrier()

        def src_ix(jh, half):
            return idx_s.at[jh, pl.ds(half * CH, CH)]

        pltpu.async_copy(h_hbm.at[src_ix(0, 0)], rows[0], gsem[0])
        pltpu.async_copy(h_hbm.at[src_ix(0, 1)], rows[1], gsem[1])
        pltpu.async_copy(h_hbm.at[src_ix(1, 0)], rows[2], gsem[2])

        def chunk(k, carry):
            for b in range(4):
                j = 4 * k + b
                jh = 2 * k + b // 2
                half = b % 2
                b3 = (b + 3) % 4
                jh3 = (4 * k + b + 3) // 2
                pltpu.make_async_copy(h_hbm.at[src_ix(jh, half)], rows[b],
                                      gsem[b]).wait()
                pltpu.async_copy(rows[b], acc.at[idx_d.at[j]], ssem[b],
                                 add=True)

                @pl.when((j + 3 < NCH) & (j >= 1))
                def _():
                    pltpu.make_async_copy(rows[b3], acc.at[idx_d.at[j - 1]],
                                          ssem[b3]).wait()

                @pl.when(j + 3 < NCH)
                def _():
                    pltpu.async_copy(h_hbm.at[src_ix(jh3, (half + 1) % 2)],
                                     rows[b3], gsem[b3])

            return carry

        lax.fori_loop(0, NCH // 4, chunk, 0)
        for j in range(NCH - 4, NCH):
            pltpu.make_async_copy(rows[j % 4], acc.at[idx_d.at[j]],
                                  ssem[j % 4]).wait()
        plsc.subcore_barrier()
        pltpu.sync_copy(acc.at[pl.ds(s * RPT, RPT)],
                        out_hbm.at[pl.ds(c * N + s * RPT, RPT)])

        @pl.when(s == 15)
        def _():
            pltpu.sync_copy(acc.at[pl.ds(REM_OFF, REM)],
                            out_hbm.at[pl.ds(c * N + REM_OFF, REM)])

    return _sc_segment_sum


def _proj_body(x_ref, w_ref, o_ref):
    o_ref[...] = jnp.dot(x_ref[...], w_ref[...],
                         preferred_element_type=jnp.float32)


def _proj(x, w):
    return pl.pallas_call(
        _proj_body,
        grid=(NB,),
        in_specs=[
            pl.BlockSpec((BLK, F_IN), lambda i: (i, 0)),
            pl.BlockSpec((F_IN, H), lambda i: (0, 0)),
        ],
        out_specs=pl.BlockSpec((BLK, H), lambda i: (i, 0)),
        out_shape=jax.ShapeDtypeStruct((N, H), jnp.float32),
    )(x, w)


def _stats_update(i, u, st_ref):
    @pl.when(i == 0)
    def _():
        st_ref[...] = jnp.zeros_like(st_ref)

    st_ref[0:1, :] += jnp.sum(u, axis=0, keepdims=True)
    st_ref[1:2, :] += jnp.sum(u * u, axis=0, keepdims=True)


def _l0_body(xp_ref, p0_ref, p1_ref, eps_ref, ba_ref, wb_ref, bb_ref,
             u_ref, st_ref):
    i = pl.program_id(0)
    t = ((1.0 + eps_ref[0, 0]) * xp_ref[...] + p0_ref[...] + p1_ref[...]
         + ba_ref[...])
    t = jnp.maximum(t, 0.0)
    u = jnp.dot(t, wb_ref[...], preferred_element_type=jnp.float32) + bb_ref[...]
    u_ref[...] = u
    _stats_update(i, u, st_ref)


def _lk_body(h_ref, p0_ref, p1_ref, eps_ref, wa_ref, ba_ref, wb_ref, bb_ref,
             u_ref, st_ref):
    i = pl.program_id(0)
    hp = (1.0 + eps_ref[0, 0]) * h_ref[...] + p0_ref[...] + p1_ref[...]
    t = jnp.maximum(
        jnp.dot(hp, wa_ref[...], preferred_element_type=jnp.float32)
        + ba_ref[...], 0.0)
    u = jnp.dot(t, wb_ref[...], preferred_element_type=jnp.float32) + bb_ref[...]
    u_ref[...] = u
    _stats_update(i, u, st_ref)


_BH = lambda i: (i, 0)
_P1 = lambda i: (NB + i, 0)
_W0 = lambda i: (0, 0)

_LAYER_OUT = dict(
    out_specs=[
        pl.BlockSpec((BLK, H), _BH),
        pl.BlockSpec((8, H), _W0),
    ],
    out_shape=[
        jax.ShapeDtypeStruct((N, H), jnp.float32),
        jax.ShapeDtypeStruct((8, H), jnp.float32),
    ],
)


def _run_layer0(xp, part, eps, ba, wb, bb):
    return pl.pallas_call(
        _l0_body,
        grid=(NB,),
        in_specs=[
            pl.BlockSpec((BLK, H), _BH),
            pl.BlockSpec((BLK, H), _BH),
            pl.BlockSpec((BLK, H), _P1),
            pl.BlockSpec(memory_space=pltpu.SMEM),
            pl.BlockSpec((1, H), _W0),
            pl.BlockSpec((H, H), _W0),
            pl.BlockSpec((1, H), _W0),
        ],
        **_LAYER_OUT,
    )(xp, part, part, eps, ba, wb, bb)


def _run_layerk(h, part, eps, wa, ba, wb, bb):
    return pl.pallas_call(
        _lk_body,
        grid=(NB,),
        in_specs=[
            pl.BlockSpec((BLK, H), _BH),
            pl.BlockSpec((BLK, H), _BH),
            pl.BlockSpec((BLK, H), _P1),
            pl.BlockSpec(memory_space=pltpu.SMEM),
            pl.BlockSpec((H, H), _W0),
            pl.BlockSpec((1, H), _W0),
            pl.BlockSpec((H, H), _W0),
            pl.BlockSpec((1, H), _W0),
        ],
        **_LAYER_OUT,
    )(h, part, part, eps, wa, ba, wb, bb)


def _bn_coeffs(st_ref, g_ref, b_ref):
    mean = st_ref[0:1, :] * (1.0 / N)
    ex2 = st_ref[1:2, :] * (1.0 / N)
    var = ex2 - mean * mean
    scale = g_ref[...] * lax.rsqrt(var + 1e-5)
    shift = b_ref[...] - mean * scale
    return scale, shift


def _norm_body(u_ref, st_ref, g_ref, b_ref, o_ref):
    scale, shift = _bn_coeffs(st_ref, g_ref, b_ref)
    o_ref[...] = jnp.maximum(u_ref[...] * scale + shift, 0.0)


def _run_norm(u, st, g, b):
    return pl.pallas_call(
        _norm_body,
        grid=(NB,),
        in_specs=[
            pl.BlockSpec((BLK, H), _BH),
            pl.BlockSpec((8, H), _W0),
            pl.BlockSpec((1, H), _W0),
            pl.BlockSpec((1, H), _W0),
        ],
        out_specs=pl.BlockSpec((BLK, H), _BH),
        out_shape=jax.ShapeDtypeStruct((N, H), jnp.float32),
    )(u, st, g, b)


def _pool_body(u_ref, st_ref, g_ref, b_ref, bat_ref, o_ref):
    i = pl.program_id(0)
    scale, shift = _bn_coeffs(st_ref, g_ref, b_ref)
    h = jnp.maximum(u_ref[...] * scale + shift, 0.0)
    onehot = (bat_ref[0] == lax.broadcasted_iota(jnp.int32, (BLK, G), 1)
              ).astype(jnp.float32)

    @pl.when(i == 0)
    def _():
        o_ref[...] = jnp.zeros_like(o_ref)

    o_ref[...] += lax.dot_general(onehot, h, (((0,), (0,)), ((), ())),
                                  preferred_element_type=jnp.float32)


def _run_pool(u, st, g, b, batch3):
    return pl.pallas_call(
        _pool_body,
        grid=(NB,),
        in_specs=[
            pl.BlockSpec((BLK, H), _BH),
            pl.BlockSpec((8, H), _W0),
            pl.BlockSpec((1, H), _W0),
            pl.BlockSpec((1, H), _W0),
            pl.BlockSpec((1, BLK, 1), lambda i: (i, 0, 0)),
        ],
        out_specs=pl.BlockSpec((G, H), _W0),
        out_shape=jax.ShapeDtypeStruct((G, H), jnp.float32),
    )(u, st, g, b, batch3)


def kernel(x, edge_index, batch, params):
    src, dst = lax.sort((edge_index[0], edge_index[1]), num_keys=1)
    ppw = EPW - E // NW
    pad_dst = jnp.broadcast_to(N + (jnp.arange(ppw, dtype=jnp.int32) % 8),
                               (NW, ppw))
    src_p = jnp.concatenate(
        [src.reshape(NW, E // NW),
         jnp.zeros((NW, ppw), jnp.int32)], axis=1).reshape(-1, 2 * CH)
    dst_p = jnp.concatenate(
        [dst.reshape(NW, E // NW), pad_dst], axis=1).reshape(-1, CH)
    zeros = jnp.zeros((RPT, H), jnp.float32)
    batch3 = batch.reshape(NB, BLK, 1)

    def r1h(v):
        return v.reshape(1, H)

    xp = _proj(x, params["W0a"])
    agg = _get_sc_agg()
    part = agg(xp, src_p, dst_p, zeros)
    u, st = _run_layer0(xp, part, params["eps0"].reshape(1, 1),
                        r1h(params["b0a"]), params["W0b"], r1h(params["b0b"]))
    h = _run_norm(u, st, r1h(params["gamma0"]), r1h(params["beta0"]))

    for i in (1, 2):
        part = agg(h, src_p, dst_p, zeros)
        u, st = _run_layerk(h, part, params[f"eps{i}"].reshape(1, 1),
                            params[f"W{i}a"], r1h(params[f"b{i}a"]),
                            params[f"W{i}b"], r1h(params[f"b{i}b"]))
        if i < 2:
            h = _run_norm(u, st, r1h(params[f"gamma{i}"]),
                          r1h(params[f"beta{i}"]))

    return _run_pool(u, st, r1h(params["gamma2"]), r1h(params["beta2"]),
                     batch3)


# speedup vs baseline: 1.4054x; 1.4054x over previous
import functools

import jax
import jax.numpy as jnp
from jax import lax
from jax.experimental import pallas as pl
from jax.experimental.pallas import tpu as pltpu
from jax.experimental.pallas import tpu_sc as plsc

N = 10000
E = 160000
F_IN = 256
H = 128
G = 128

NW = 32
CH = 64
EPW = 5120
E_PAD = NW * EPW
NCH = EPW // CH
RPT = 624
REM = N - 16 * RPT
REM_OFF = 16 * RPT
N_ACC = N + 8
BLK = 1000
NB = N // BLK

@functools.cache
def _get_sc_agg():
    mesh = plsc.VectorSubcoreMesh(core_axis_name="c", subcore_axis_name="s")

    @functools.partial(
        pl.kernel,
        out_type=jax.ShapeDtypeStruct((2 * N, H), jnp.float32),
        mesh=mesh,
        scratch_types=[
            pltpu.VMEM((NCH // 2, 2 * CH), jnp.int32),
            pltpu.VMEM((NCH, CH), jnp.int32),
            [pltpu.VMEM((CH, H), jnp.float32)] * 4,
            pltpu.VMEM_SHARED((N_ACC, H), jnp.float32),
            [pltpu.SemaphoreType.DMA] * 4,
            [pltpu.SemaphoreType.DMA] * 4,
        ],
    )
    def _sc_segment_sum(h_hbm, src_hbm, dst_hbm, zeros_hbm, out_hbm,
                        idx_s, idx_d, rows, acc, gsem, ssem):
        c = lax.axis_index("c")
        s = lax.axis_index("s")
        w = s * 2 + c
        pltpu.sync_copy(zeros_hbm, acc.at[pl.ds(s * RPT, RPT)])
        pltpu.sync_copy(src_hbm.at[pl.ds(w * (NCH // 2), NCH // 2)], idx_s)
        pltpu.sync_copy(dst_hbm.at[pl.ds(w * NCH, NCH)], idx_d)

        @pl.when(s == 15)
        def _():
            pltpu.sync_copy(zeros_hbm.at[pl.ds(0, REM)],
                            acc.at[pl.ds(REM_OFF, REM)])

        plsc.subcore_barrier()

        def src_ix(jh, half):
            return idx_s.at[jh, pl.ds(half * CH, CH)]

        pltpu.async_copy(h_hbm.at[src_ix(0, 0)], rows[0], gsem[0])
        pltpu.async_copy(h_hbm.at[src_ix(0, 1)], rows[1], gsem[1])
        pltpu.async_copy(h_hbm.at[src_ix(1, 0)], rows[2], gsem[2])

        def chunk(k, carry):
            for b in range(4):
                j = 4 * k + b
                jh = 2 * k + b // 2
                half = b % 2
                b3 = (b + 3) % 4
                jh3 = (4 * k + b + 3) // 2
                pltpu.make_async_copy(h_hbm.at[src_ix(jh, half)], rows[b],
                                      gsem[b]).wait()
                pltpu.async_copy(rows[b], acc.at[idx_d.at[j]], ssem[b],
                                 add=True)

                @pl.when((j + 3 < NCH) & (j >= 1))
                def _():
                    pltpu.make_async_copy(rows[b3], acc.at[idx_d.at[j - 1]],
                                          ssem[b3]).wait()

                @pl.when(j + 3 < NCH)
                def _():
                    pltpu.async_copy(h_hbm.at[src_ix(jh3, (half + 1) % 2)],
                                     rows[b3], gsem[b3])

            return carry

        lax.fori_loop(0, NCH // 4, chunk, 0)
        for j in range(NCH - 4, NCH):
            pltpu.make_async_copy(rows[j % 4], acc.at[idx_d.at[j]],
                                  ssem[j % 4]).wait()
        plsc.subcore_barrier()
        pltpu.sync_copy(acc.at[pl.ds(s * RPT, RPT)],
                        out_hbm.at[pl.ds(c * N + s * RPT, RPT)])

        @pl.when(s == 15)
        def _():
            pltpu.sync_copy(acc.at[pl.ds(REM_OFF, REM)],
                            out_hbm.at[pl.ds(c * N + REM_OFF, REM)])

    return _sc_segment_sum


def _proj_body(x_ref, w_ref, o_ref):
    o_ref[...] = jnp.dot(x_ref[...], w_ref[...],
                         preferred_element_type=jnp.float32)


def _proj(x, w):
    return pl.pallas_call(
        _proj_body,
        grid=(NB,),
        in_specs=[
            pl.BlockSpec((BLK, F_IN), lambda i: (i, 0)),
            pl.BlockSpec((F_IN, H), lambda i: (0, 0)),
        ],
        out_specs=pl.BlockSpec((BLK, H), lambda i: (i, 0)),
        out_shape=jax.ShapeDtypeStruct((N, H), jnp.float32),
    )(x, w)


def _stats_update(i, u, st_ref):
    @pl.when(i == 0)
    def _():
        st_ref[...] = jnp.zeros_like(st_ref)

    st_ref[0:1, :] += jnp.sum(u, axis=0, keepdims=True)
    st_ref[1:2, :] += jnp.sum(u * u, axis=0, keepdims=True)


def _l0_body(xp_ref, p0_ref, p1_ref, eps_ref, ba_ref, wb_ref, bb_ref,
             u_ref, st_ref):
    i = pl.program_id(0)
    t = ((1.0 + eps_ref[0, 0]) * xp_ref[...] + p0_ref[...] + p1_ref[...]
         + ba_ref[...])
    t = jnp.maximum(t, 0.0)
    u = jnp.dot(t, wb_ref[...], preferred_element_type=jnp.float32) + bb_ref[...]
    u_ref[...] = u
    _stats_update(i, u, st_ref)


def _lk_body(h_ref, p0_ref, p1_ref, eps_ref, wa_ref, ba_ref, wb_ref, bb_ref,
             u_ref, st_ref):
    i = pl.program_id(0)
    hp = (1.0 + eps_ref[0, 0]) * h_ref[...] + p0_ref[...] + p1_ref[...]
    t = jnp.maximum(
        jnp.dot(hp, wa_ref[...], preferred_element_type=jnp.float32)
        + ba_ref[...], 0.0)
    u = jnp.dot(t, wb_ref[...], preferred_element_type=jnp.float32) + bb_ref[...]
    u_ref[...] = u
    _stats_update(i, u, st_ref)


_BH = lambda i: (i, 0)
_P1 = lambda i: (NB + i, 0)
_W0 = lambda i: (0, 0)

_LAYER_OUT = dict(
    out_specs=[
        pl.BlockSpec((BLK, H), _BH),
        pl.BlockSpec((8, H), _W0),
    ],
    out_shape=[
        jax.ShapeDtypeStruct((N, H), jnp.float32),
        jax.ShapeDtypeStruct((8, H), jnp.float32),
    ],
)


def _run_layer0(xp, part, eps, ba, wb, bb):
    return pl.pallas_call(
        _l0_body,
        grid=(NB,),
        in_specs=[
            pl.BlockSpec((BLK, H), _BH),
            pl.BlockSpec((BLK, H), _BH),
            pl.BlockSpec((BLK, H), _P1),
            pl.BlockSpec(memory_space=pltpu.SMEM),
            pl.BlockSpec((1, H), _W0),
            pl.BlockSpec((H, H), _W0),
            pl.BlockSpec((1, H), _W0),
        ],
        **_LAYER_OUT,
    )(xp, part, part, eps, ba, wb, bb)


def _run_layerk(h, part, eps, wa, ba, wb, bb):
    return pl.pallas_call(
        _lk_body,
        grid=(NB,),
        in_specs=[
            pl.BlockSpec((BLK, H), _BH),
            pl.BlockSpec((BLK, H), _BH),
            pl.BlockSpec((BLK, H), _P1),
            pl.BlockSpec(memory_space=pltpu.SMEM),
            pl.BlockSpec((H, H), _W0),
            pl.BlockSpec((1, H), _W0),
            pl.BlockSpec((H, H), _W0),
            pl.BlockSpec((1, H), _W0),
        ],
        **_LAYER_OUT,
    )(h, part, part, eps, wa, ba, wb, bb)


def _bn_coeffs(st_ref, g_ref, b_ref):
    mean = st_ref[0:1, :] * (1.0 / N)
    ex2 = st_ref[1:2, :] * (1.0 / N)
    var = ex2 - mean * mean
    scale = g_ref[...] * lax.rsqrt(var + 1e-5)
    shift = b_ref[...] - mean * scale
    return scale, shift


def _norm_body(u_ref, st_ref, g_ref, b_ref, o_ref):
    scale, shift = _bn_coeffs(st_ref, g_ref, b_ref)
    o_ref[...] = jnp.maximum(u_ref[...] * scale + shift, 0.0)


def _run_norm(u, st, g, b):
    return pl.pallas_call(
        _norm_body,
        grid=(NB,),
        in_specs=[
            pl.BlockSpec((BLK, H), _BH),
            pl.BlockSpec((8, H), _W0),
            pl.BlockSpec((1, H), _W0),
            pl.BlockSpec((1, H), _W0),
        ],
        out_specs=pl.BlockSpec((BLK, H), _BH),
        out_shape=jax.ShapeDtypeStruct((N, H), jnp.float32),
    )(u, st, g, b)


def _pool_body(u_ref, st_ref, g_ref, b_ref, bat_ref, o_ref):
    i = pl.program_id(0)
    scale, shift = _bn_coeffs(st_ref, g_ref, b_ref)
    h = jnp.maximum(u_ref[...] * scale + shift, 0.0)
    onehot = (bat_ref[0] == lax.broadcasted_iota(jnp.int32, (BLK, G), 1)
              ).astype(jnp.float32)

    @pl.when(i == 0)
    def _():
        o_ref[...] = jnp.zeros_like(o_ref)

    o_ref[...] += lax.dot_general(onehot, h, (((0,), (0,)), ((), ())),
                                  preferred_element_type=jnp.float32)


def _run_pool(u, st, g, b, batch3):
    return pl.pallas_call(
        _pool_body,
        grid=(NB,),
        in_specs=[
            pl.BlockSpec((BLK, H), _BH),
            pl.BlockSpec((8, H), _W0),
            pl.BlockSpec((1, H), _W0),
            pl.BlockSpec((1, H), _W0),
            pl.BlockSpec((1, BLK, 1), lambda i: (i, 0, 0)),
        ],
        out_specs=pl.BlockSpec((G, H), _W0),
        out_shape=jax.ShapeDtypeStruct((G, H), jnp.float32),
    )(u, st, g, b, batch3)


def kernel(x, edge_index, batch, params):
    src = edge_index[0]
    dst = edge_index[1]
    ppw = EPW - E // NW
    pad_dst = jnp.broadcast_to(N + (jnp.arange(ppw, dtype=jnp.int32) % 8),
                               (NW, ppw))
    src_p = jnp.concatenate(
        [src.reshape(NW, E // NW),
         jnp.zeros((NW, ppw), jnp.int32)], axis=1).reshape(-1, 2 * CH)
    dst_p = jnp.concatenate(
        [dst.reshape(NW, E // NW), pad_dst], axis=1).reshape(-1, CH)
    zeros = jnp.zeros((RPT, H), jnp.float32)
    batch3 = batch.reshape(NB, BLK, 1)

    def r1h(v):
        return v.reshape(1, H)

    xp = _proj(x, params["W0a"])
    agg = _get_sc_agg()
    part = agg(xp, src_p, dst_p, zeros)
    u, st = _run_layer0(xp, part, params["eps0"].reshape(1, 1),
                        r1h(params["b0a"]), params["W0b"], r1h(params["b0b"]))
    h = _run_norm(u, st, r1h(params["gamma0"]), r1h(params["beta0"]))

    for i in (1, 2):
        part = agg(h, src_p, dst_p, zeros)
        u, st = _run_layerk(h, part, params[f"eps{i}"].reshape(1, 1),
                            params[f"W{i}a"], r1h(params[f"b{i}a"]),
                            params[f"W{i}b"], r1h(params[f"b{i}b"]))
        if i < 2:
            h = _run_norm(u, st, r1h(params[f"gamma{i}"]),
                          r1h(params[f"beta{i}"]))

    return _run_pool(u, st, r1h(params["gamma2"]), r1h(params["beta2"]),
                     batch3)


# speedup vs baseline: 1.4405x; 1.0250x over previous
import functools

import jax
import jax.numpy as jnp
from jax import lax
from jax.experimental import pallas as pl
from jax.experimental.pallas import tpu as pltpu
from jax.experimental.pallas import tpu_sc as plsc

N = 10000
E = 160000
F_IN = 256
H = 128
G = 128

NW = 32
CH = 64
EPW = 5120
E_PAD = NW * EPW
NCH = EPW // CH
RPT = 624
REM = N - 16 * RPT
REM_OFF = 16 * RPT
N_ACC = N + 8
BLK = 2000
NB = N // BLK

@functools.cache
def _get_sc_agg():
    mesh = plsc.VectorSubcoreMesh(core_axis_name="c", subcore_axis_name="s")

    @functools.partial(
        pl.kernel,
        out_type=jax.ShapeDtypeStruct((2 * N, H), jnp.float32),
        mesh=mesh,
        scratch_types=[
            pltpu.VMEM((NCH // 2, 2 * CH), jnp.int32),
            pltpu.VMEM((NCH, CH), jnp.int32),
            [pltpu.VMEM((CH, H), jnp.float32)] * 4,
            pltpu.VMEM_SHARED((N_ACC, H), jnp.float32),
            [pltpu.SemaphoreType.DMA] * 4,
            [pltpu.SemaphoreType.DMA] * 4,
        ],
    )
    def _sc_segment_sum(h_hbm, src_hbm, dst_hbm, zeros_hbm, out_hbm,
                        idx_s, idx_d, rows, acc, gsem, ssem):
        c = lax.axis_index("c")
        s = lax.axis_index("s")
        w = s * 2 + c
        pltpu.sync_copy(zeros_hbm, acc.at[pl.ds(s * RPT, RPT)])
        pltpu.sync_copy(src_hbm.at[pl.ds(w * (NCH // 2), NCH // 2)], idx_s)
        pltpu.sync_copy(dst_hbm.at[pl.ds(w * NCH, NCH)], idx_d)

        @pl.when(s == 15)
        def _():
            pltpu.sync_copy(zeros_hbm.at[pl.ds(0, REM)],
                            acc.at[pl.ds(REM_OFF, REM)])

        plsc.subcore_barrier()

        def src_ix(jh, half):
            return idx_s.at[jh, pl.ds(half * CH, CH)]

        pltpu.async_copy(h_hbm.at[src_ix(0, 0)], rows[0], gsem[0])
        pltpu.async_copy(h_hbm.at[src_ix(0, 1)], rows[1], gsem[1])
        pltpu.async_copy(h_hbm.at[src_ix(1, 0)], rows[2], gsem[2])

        def chunk(k, carry):
            for b in range(4):
                j = 4 * k + b
                jh = 2 * k + b // 2
                half = b % 2
                b3 = (b + 3) % 4
                jh3 = (4 * k + b + 3) // 2
                pltpu.make_async_copy(h_hbm.at[src_ix(jh, half)], rows[b],
                                      gsem[b]).wait()
                pltpu.async_copy(rows[b], acc.at[idx_d.at[j]], ssem[b],
                                 add=True)

                @pl.when((j + 3 < NCH) & (j >= 1))
                def _():
                    pltpu.make_async_copy(rows[b3], acc.at[idx_d.at[j - 1]],
                                          ssem[b3]).wait()

                @pl.when(j + 3 < NCH)
                def _():
                    pltpu.async_copy(h_hbm.at[src_ix(jh3, (half + 1) % 2)],
                                     rows[b3], gsem[b3])

            return carry

        lax.fori_loop(0, NCH // 4, chunk, 0)
        for j in range(NCH - 4, NCH):
            pltpu.make_async_copy(rows[j % 4], acc.at[idx_d.at[j]],
                                  ssem[j % 4]).wait()
        plsc.subcore_barrier()
        pltpu.sync_copy(acc.at[pl.ds(s * RPT, RPT)],
                        out_hbm.at[pl.ds(c * N + s * RPT, RPT)])

        @pl.when(s == 15)
        def _():
            pltpu.sync_copy(acc.at[pl.ds(REM_OFF, REM)],
                            out_hbm.at[pl.ds(c * N + REM_OFF, REM)])

    return _sc_segment_sum


def _proj_body(x_ref, w_ref, o_ref):
    o_ref[...] = jnp.dot(x_ref[...], w_ref[...],
                         preferred_element_type=jnp.float32)


def _proj(x, w):
    return pl.pallas_call(
        _proj_body,
        grid=(NB,),
        in_specs=[
            pl.BlockSpec((BLK, F_IN), lambda i: (i, 0)),
            pl.BlockSpec((F_IN, H), lambda i: (0, 0)),
        ],
        out_specs=pl.BlockSpec((BLK, H), lambda i: (i, 0)),
        out_shape=jax.ShapeDtypeStruct((N, H), jnp.float32),
    )(x, w)


def _stats_update(i, u, st_ref):
    @pl.when(i == 0)
    def _():
        st_ref[...] = jnp.zeros_like(st_ref)

    st_ref[0:1, :] += jnp.sum(u, axis=0, keepdims=True)
    st_ref[1:2, :] += jnp.sum(u * u, axis=0, keepdims=True)


def _l0_body(xp_ref, p0_ref, p1_ref, eps_ref, ba_ref, wb_ref, bb_ref,
             u_ref, st_ref):
    i = pl.program_id(0)
    t = ((1.0 + eps_ref[0, 0]) * xp_ref[...] + p0_ref[...] + p1_ref[...]
         + ba_ref[...])
    t = jnp.maximum(t, 0.0)
    u = jnp.dot(t, wb_ref[...], preferred_element_type=jnp.float32) + bb_ref[...]
    u_ref[...] = u
    _stats_update(i, u, st_ref)


def _lk_body(h_ref, p0_ref, p1_ref, eps_ref, wa_ref, ba_ref, wb_ref, bb_ref,
             u_ref, st_ref):
    i = pl.program_id(0)
    hp = (1.0 + eps_ref[0, 0]) * h_ref[...] + p0_ref[...] + p1_ref[...]
    t = jnp.maximum(
        jnp.dot(hp, wa_ref[...], preferred_element_type=jnp.float32)
        + ba_ref[...], 0.0)
    u = jnp.dot(t, wb_ref[...], preferred_element_type=jnp.float32) + bb_ref[...]
    u_ref[...] = u
    _stats_update(i, u, st_ref)


_BH = lambda i: (i, 0)
_P1 = lambda i: (NB + i, 0)
_W0 = lambda i: (0, 0)

_LAYER_OUT = dict(
    out_specs=[
        pl.BlockSpec((BLK, H), _BH),
        pl.BlockSpec((8, H), _W0),
    ],
    out_shape=[
        jax.ShapeDtypeStruct((N, H), jnp.float32),
        jax.ShapeDtypeStruct((8, H), jnp.float32),
    ],
)


def _run_layer0(xp, part, eps, ba, wb, bb):
    return pl.pallas_call(
        _l0_body,
        grid=(NB,),
        in_specs=[
            pl.BlockSpec((BLK, H), _BH),
            pl.BlockSpec((BLK, H), _BH),
            pl.BlockSpec((BLK, H), _P1),
            pl.BlockSpec(memory_space=pltpu.SMEM),
            pl.BlockSpec((1, H), _W0),
            pl.BlockSpec((H, H), _W0),
            pl.BlockSpec((1, H), _W0),
        ],
        **_LAYER_OUT,
    )(xp, part, part, eps, ba, wb, bb)


def _run_layerk(h, part, eps, wa, ba, wb, bb):
    return pl.pallas_call(
        _lk_body,
        grid=(NB,),
        in_specs=[
            pl.BlockSpec((BLK, H), _BH),
            pl.BlockSpec((BLK, H), _BH),
            pl.BlockSpec((BLK, H), _P1),
            pl.BlockSpec(memory_space=pltpu.SMEM),
            pl.BlockSpec((H, H), _W0),
            pl.BlockSpec((1, H), _W0),
            pl.BlockSpec((H, H), _W0),
            pl.BlockSpec((1, H), _W0),
        ],
        **_LAYER_OUT,
    )(h, part, part, eps, wa, ba, wb, bb)


def _bn_coeffs(st_ref, g_ref, b_ref):
    mean = st_ref[0:1, :] * (1.0 / N)
    ex2 = st_ref[1:2, :] * (1.0 / N)
    var = ex2 - mean * mean
    scale = g_ref[...] * lax.rsqrt(var + 1e-5)
    shift = b_ref[...] - mean * scale
    return scale, shift


def _norm_body(u_ref, st_ref, g_ref, b_ref, o_ref):
    scale, shift = _bn_coeffs(st_ref, g_ref, b_ref)
    o_ref[...] = jnp.maximum(u_ref[...] * scale + shift, 0.0)


def _run_norm(u, st, g, b):
    return pl.pallas_call(
        _norm_body,
        grid=(NB,),
        in_specs=[
            pl.BlockSpec((BLK, H), _BH),
            pl.BlockSpec((8, H), _W0),
            pl.BlockSpec((1, H), _W0),
            pl.BlockSpec((1, H), _W0),
        ],
        out_specs=pl.BlockSpec((BLK, H), _BH),
        out_shape=jax.ShapeDtypeStruct((N, H), jnp.float32),
    )(u, st, g, b)


def _pool_body(u_ref, st_ref, g_ref, b_ref, bat_ref, o_ref):
    i = pl.program_id(0)
    scale, shift = _bn_coeffs(st_ref, g_ref, b_ref)
    h = jnp.maximum(u_ref[...] * scale + shift, 0.0)
    onehot = (bat_ref[0] == lax.broadcasted_iota(jnp.int32, (BLK, G), 1)
              ).astype(jnp.float32)

    @pl.when(i == 0)
    def _():
        o_ref[...] = jnp.zeros_like(o_ref)

    o_ref[...] += lax.dot_general(onehot, h, (((0,), (0,)), ((), ())),
                                  preferred_element_type=jnp.float32)


def _run_pool(u, st, g, b, batch3):
    return pl.pallas_call(
        _pool_body,
        grid=(NB,),
        in_specs=[
            pl.BlockSpec((BLK, H), _BH),
            pl.BlockSpec((8, H), _W0),
            pl.BlockSpec((1, H), _W0),
            pl.BlockSpec((1, H), _W0),
            pl.BlockSpec((1, BLK, 1), lambda i: (i, 0, 0)),
        ],
        out_specs=pl.BlockSpec((G, H), _W0),
        out_shape=jax.ShapeDtypeStruct((G, H), jnp.float32),
    )(u, st, g, b, batch3)


def kernel(x, edge_index, batch, params):
    src = edge_index[0]
    dst = edge_index[1]
    ppw = EPW - E // NW
    pad_dst = jnp.broadcast_to(N + (jnp.arange(ppw, dtype=jnp.int32) % 8),
                               (NW, ppw))
    src_p = jnp.concatenate(
        [src.reshape(NW, E // NW),
         jnp.zeros((NW, ppw), jnp.int32)], axis=1).reshape(-1, 2 * CH)
    dst_p = jnp.concatenate(
        [dst.reshape(NW, E // NW), pad_dst], axis=1).reshape(-1, CH)
    zeros = jnp.zeros((RPT, H), jnp.float32)
    batch3 = batch.reshape(NB, BLK, 1)

    def r1h(v):
        return v.reshape(1, H)

    xp = _proj(x, params["W0a"])
    agg = _get_sc_agg()
    part = agg(xp, src_p, dst_p, zeros)
    u, st = _run_layer0(xp, part, params["eps0"].reshape(1, 1),
                        r1h(params["b0a"]), params["W0b"], r1h(params["b0b"]))
    h = _run_norm(u, st, r1h(params["gamma0"]), r1h(params["beta0"]))

    for i in (1, 2):
        part = agg(h, src_p, dst_p, zeros)
        u, st = _run_layerk(h, part, params[f"eps{i}"].reshape(1, 1),
                            params[f"W{i}a"], r1h(params[f"b{i}a"]),
                            params[f"W{i}b"], r1h(params[f"b{i}b"]))
        if i < 2:
            h = _run_norm(u, st, r1h(params[f"gamma{i}"]),
                          r1h(params[f"beta{i}"]))

    return _run_pool(u, st, r1h(params["gamma2"]), r1h(params["beta2"]),
                     batch3)


# speedup vs baseline: 1.4440x; 1.0025x over previous
import functools

import jax
import jax.numpy as jnp
from jax import lax
from jax.experimental import pallas as pl
from jax.experimental.pallas import tpu as pltpu
from jax.experimental.pallas import tpu_sc as plsc

N = 10000
E = 160000
F_IN = 256
H = 128
G = 128

NW = 32
CH = 64
EPW = 5120
E_PAD = NW * EPW
NCH = EPW // CH
RPT = 624
REM = N - 16 * RPT
REM_OFF = 16 * RPT
N_ACC = N + 8
BLK = 2000
NB = N // BLK

@functools.cache
def _get_sc_agg():
    mesh = plsc.VectorSubcoreMesh(core_axis_name="c", subcore_axis_name="s")

    @functools.partial(
        pl.kernel,
        out_type=jax.ShapeDtypeStruct((2 * N, H), jnp.float32),
        mesh=mesh,
        scratch_types=[
            pltpu.VMEM((NCH // 2, 2 * CH), jnp.int32),
            pltpu.VMEM((NCH, CH), jnp.int32),
            [pltpu.VMEM((CH, H), jnp.float32)] * 4,
            pltpu.VMEM_SHARED((N_ACC, H), jnp.float32),
            [pltpu.SemaphoreType.DMA] * 4,
            [pltpu.SemaphoreType.DMA] * 4,
        ],
    )
    def _sc_segment_sum(h_hbm, src_hbm, dst_hbm, zeros_hbm, out_hbm,
                        idx_s, idx_d, rows, acc, gsem, ssem):
        c = lax.axis_index("c")
        s = lax.axis_index("s")
        w = s * 2 + c
        def src_ix(jh, half):
            return idx_s.at[jh, pl.ds(half * CH, CH)]

        pltpu.sync_copy(src_hbm.at[pl.ds(w * (NCH // 2), NCH // 2)], idx_s)
        pltpu.async_copy(h_hbm.at[src_ix(0, 0)], rows[0], gsem[0])
        pltpu.async_copy(h_hbm.at[src_ix(0, 1)], rows[1], gsem[1])
        pltpu.async_copy(h_hbm.at[src_ix(1, 0)], rows[2], gsem[2])
        pltpu.sync_copy(dst_hbm.at[pl.ds(w * NCH, NCH)], idx_d)
        pltpu.sync_copy(zeros_hbm, acc.at[pl.ds(s * RPT, RPT)])

        @pl.when(s == 15)
        def _():
            pltpu.sync_copy(zeros_hbm.at[pl.ds(0, REM)],
                            acc.at[pl.ds(REM_OFF, REM)])

        plsc.subcore_barrier()


        def chunk(k, carry):
            for b in range(4):
                j = 4 * k + b
                jh = 2 * k + b // 2
                half = b % 2
                b3 = (b + 3) % 4
                jh3 = (4 * k + b + 3) // 2
                pltpu.make_async_copy(h_hbm.at[src_ix(jh, half)], rows[b],
                                      gsem[b]).wait()
                pltpu.async_copy(rows[b], acc.at[idx_d.at[j]], ssem[b],
                                 add=True)

                @pl.when((j + 3 < NCH) & (j >= 1))
                def _():
                    pltpu.make_async_copy(rows[b3], acc.at[idx_d.at[j - 1]],
                                          ssem[b3]).wait()

                @pl.when(j + 3 < NCH)
                def _():
                    pltpu.async_copy(h_hbm.at[src_ix(jh3, (half + 1) % 2)],
                                     rows[b3], gsem[b3])

            return carry

        lax.fori_loop(0, NCH // 4, chunk, 0)
        for j in range(NCH - 4, NCH):
            pltpu.make_async_copy(rows[j % 4], acc.at[idx_d.at[j]],
                                  ssem[j % 4]).wait()
        plsc.subcore_barrier()
        pltpu.sync_copy(acc.at[pl.ds(s * RPT, RPT)],
                        out_hbm.at[pl.ds(c * N + s * RPT, RPT)])

        @pl.when(s == 15)
        def _():
            pltpu.sync_copy(acc.at[pl.ds(REM_OFF, REM)],
                            out_hbm.at[pl.ds(c * N + REM_OFF, REM)])

    return _sc_segment_sum


def _proj_body(x_ref, w_ref, o_ref):
    o_ref[...] = jnp.dot(x_ref[...], w_ref[...],
                         preferred_element_type=jnp.float32)


def _proj(x, w):
    return pl.pallas_call(
        _proj_body,
        grid=(NB,),
        in_specs=[
            pl.BlockSpec((BLK, F_IN), lambda i: (i, 0)),
            pl.BlockSpec((F_IN, H), lambda i: (0, 0)),
        ],
        out_specs=pl.BlockSpec((BLK, H), lambda i: (i, 0)),
        out_shape=jax.ShapeDtypeStruct((N, H), jnp.float32),
    )(x, w)


def _stats_update(i, u, st_ref):
    @pl.when(i == 0)
    def _():
        st_ref[...] = jnp.zeros_like(st_ref)

    st_ref[0:1, :] += jnp.sum(u, axis=0, keepdims=True)
    st_ref[1:2, :] += jnp.sum(u * u, axis=0, keepdims=True)


def _l0_body(xp_ref, p0_ref, p1_ref, eps_ref, ba_ref, wb_ref, bb_ref,
             u_ref, st_ref):
    i = pl.program_id(0)
    t = ((1.0 + eps_ref[0, 0]) * xp_ref[...] + p0_ref[...] + p1_ref[...]
         + ba_ref[...])
    t = jnp.maximum(t, 0.0)
    u = jnp.dot(t, wb_ref[...], preferred_element_type=jnp.float32) + bb_ref[...]
    u_ref[...] = u
    _stats_update(i, u, st_ref)


def _lk_body(h_ref, p0_ref, p1_ref, eps_ref, wa_ref, ba_ref, wb_ref, bb_ref,
             u_ref, st_ref):
    i = pl.program_id(0)
    hp = (1.0 + eps_ref[0, 0]) * h_ref[...] + p0_ref[...] + p1_ref[...]
    t = jnp.maximum(
        jnp.dot(hp, wa_ref[...], preferred_element_type=jnp.float32)
        + ba_ref[...], 0.0)
    u = jnp.dot(t, wb_ref[...], preferred_element_type=jnp.float32) + bb_ref[...]
    u_ref[...] = u
    _stats_update(i, u, st_ref)


_BH = lambda i: (i, 0)
_P1 = lambda i: (NB + i, 0)
_W0 = lambda i: (0, 0)

_LAYER_OUT = dict(
    out_specs=[
        pl.BlockSpec((BLK, H), _BH),
        pl.BlockSpec((8, H), _W0),
    ],
    out_shape=[
        jax.ShapeDtypeStruct((N, H), jnp.float32),
        jax.ShapeDtypeStruct((8, H), jnp.float32),
    ],
)


def _run_layer0(xp, part, eps, ba, wb, bb):
    return pl.pallas_call(
        _l0_body,
        grid=(NB,),
        in_specs=[
            pl.BlockSpec((BLK, H), _BH),
            pl.BlockSpec((BLK, H), _BH),
            pl.BlockSpec((BLK, H), _P1),
            pl.BlockSpec(memory_space=pltpu.SMEM),
            pl.BlockSpec((1, H), _W0),
            pl.BlockSpec((H, H), _W0),
            pl.BlockSpec((1, H), _W0),
        ],
        **_LAYER_OUT,
    )(xp, part, part, eps, ba, wb, bb)


def _run_layerk(h, part, eps, wa, ba, wb, bb):
    return pl.pallas_call(
        _lk_body,
        grid=(NB,),
        in_specs=[
            pl.BlockSpec((BLK, H), _BH),
            pl.BlockSpec((BLK, H), _BH),
            pl.BlockSpec((BLK, H), _P1),
            pl.BlockSpec(memory_space=pltpu.SMEM),
            pl.BlockSpec((H, H), _W0),
            pl.BlockSpec((1, H), _W0),
            pl.BlockSpec((H, H), _W0),
            pl.BlockSpec((1, H), _W0),
        ],
        **_LAYER_OUT,
    )(h, part, part, eps, wa, ba, wb, bb)


def _bn_coeffs(st_ref, g_ref, b_ref):
    mean = st_ref[0:1, :] * (1.0 / N)
    ex2 = st_ref[1:2, :] * (1.0 / N)
    var = ex2 - mean * mean
    scale = g_ref[...] * lax.rsqrt(var + 1e-5)
    shift = b_ref[...] - mean * scale
    return scale, shift


def _norm_body(u_ref, st_ref, g_ref, b_ref, o_ref):
    scale, shift = _bn_coeffs(st_ref, g_ref, b_ref)
    o_ref[...] = jnp.maximum(u_ref[...] * scale + shift, 0.0)


def _run_norm(u, st, g, b):
    return pl.pallas_call(
        _norm_body,
        grid=(NB,),
        in_specs=[
            pl.BlockSpec((BLK, H), _BH),
            pl.BlockSpec((8, H), _W0),
            pl.BlockSpec((1, H), _W0),
            pl.BlockSpec((1, H), _W0),
        ],
        out_specs=pl.BlockSpec((BLK, H), _BH),
        out_shape=jax.ShapeDtypeStruct((N, H), jnp.float32),
    )(u, st, g, b)


def _pool_body(u_ref, st_ref, g_ref, b_ref, bat_ref, o_ref):
    i = pl.program_id(0)
    scale, shift = _bn_coeffs(st_ref, g_ref, b_ref)
    h = jnp.maximum(u_ref[...] * scale + shift, 0.0)
    onehot = (bat_ref[0] == lax.broadcasted_iota(jnp.int32, (BLK, G), 1)
              ).astype(jnp.float32)

    @pl.when(i == 0)
    def _():
        o_ref[...] = jnp.zeros_like(o_ref)

    o_ref[...] += lax.dot_general(onehot, h, (((0,), (0,)), ((), ())),
                                  preferred_element_type=jnp.float32)


def _run_pool(u, st, g, b, batch3):
    return pl.pallas_call(
        _pool_body,
        grid=(NB,),
        in_specs=[
            pl.BlockSpec((BLK, H), _BH),
            pl.BlockSpec((8, H), _W0),
            pl.BlockSpec((1, H), _W0),
            pl.BlockSpec((1, H), _W0),
            pl.BlockSpec((1, BLK, 1), lambda i: (i, 0, 0)),
        ],
        out_specs=pl.BlockSpec((G, H), _W0),
        out_shape=jax.ShapeDtypeStruct((G, H), jnp.float32),
    )(u, st, g, b, batch3)


def kernel(x, edge_index, batch, params):
    src = edge_index[0]
    dst = edge_index[1]
    ppw = EPW - E // NW
    pad_dst = jnp.broadcast_to(N + (jnp.arange(ppw, dtype=jnp.int32) % 8),
                               (NW, ppw))
    src_p = jnp.concatenate(
        [src.reshape(NW, E // NW),
         jnp.zeros((NW, ppw), jnp.int32)], axis=1).reshape(-1, 2 * CH)
    dst_p = jnp.concatenate(
        [dst.reshape(NW, E // NW), pad_dst], axis=1).reshape(-1, CH)
    zeros = jnp.zeros((RPT, H), jnp.float32)
    batch3 = batch.reshape(NB, BLK, 1)

    def r1h(v):
        return v.reshape(1, H)

    xp = _proj(x, params["W0a"])
    agg = _get_sc_agg()
    part = agg(xp, src_p, dst_p, zeros)
    u, st = _run_layer0(xp, part, params["eps0"].reshape(1, 1),
                        r1h(params["b0a"]), params["W0b"], r1h(params["b0b"]))
    h = _run_norm(u, st, r1h(params["gamma0"]), r1h(params["beta0"]))

    for i in (1, 2):
        part = agg(h, src_p, dst_p, zeros)
        u, st = _run_layerk(h, part, params[f"eps{i}"].reshape(1, 1),
                            params[f"W{i}a"], r1h(params[f"b{i}a"]),
                            params[f"W{i}b"], r1h(params[f"b{i}b"]))
        if i < 2:
            h = _run_norm(u, st, r1h(params[f"gamma{i}"]),
                          r1h(params[f"beta{i}"]))

    return _run_pool(u, st, r1h(params["gamma2"]), r1h(params["beta2"]),
                     batch3)
